# baseline (speedup 1.0000x reference)
import functools

import jax
import jax.numpy as jnp
from jax import lax
from jax.experimental import pallas as pl
from jax.experimental.pallas import tpu as pltpu
from jax.experimental.pallas import tpu_sc as plsc

H = 128
NC, NS, L = 2, 16, 16
NW = NC * NS
WAVE = 128
EALIGN = NW * WAVE
GROWS = 256
NSLOPE = 0.2

f32 = jnp.float32
i32 = jnp.int32


def _rup(n, m):
    return ((n + m - 1) // m) * m


def _mesh():
    return plsc.VectorSubcoreMesh(core_axis_name="c", subcore_axis_name="s",
                                  num_cores=NC, num_subcores=NS)


_BR = 256


def _tc_call(body, nrows, out_width, n_in, extra_specs, out_shapes=None):
    grid = (nrows // _BR,)
    if out_shapes is None:
        out_shapes = jax.ShapeDtypeStruct((nrows, out_width), f32)
        out_specs = pl.BlockSpec((_BR, out_width), lambda i: (i, 0))
    else:
        out_specs = [pl.BlockSpec((_BR, s.shape[1]), lambda i: (i, 0))
                     for s in out_shapes]
    return pl.pallas_call(
        body, grid=grid,
        in_specs=extra_specs,
        out_specs=out_specs,
        out_shape=out_shapes,
    )


def tc_mm(x, w, b):
    def body(x_ref, w_ref, b_ref, o_ref):
        o_ref[...] = jnp.dot(x_ref[...], w_ref[...],
                             preferred_element_type=f32) + b_ref[...]
    N = x.shape[0]
    specs = [pl.BlockSpec((_BR, H), lambda i: (i, 0)),
             pl.BlockSpec((H, H), lambda i: (0, 0)),
             pl.BlockSpec((1, H), lambda i: (0, 0))]
    return _tc_call(body, N, H, 3, specs)(x, w, b)


def tc_mm2(x, y, wx, wy, b, relu):
    def body(x_ref, y_ref, wx_ref, wy_ref, b_ref, o_ref):
        acc = jnp.dot(x_ref[...], wx_ref[...], preferred_element_type=f32)
        acc = acc + jnp.dot(y_ref[...], wy_ref[...],
                            preferred_element_type=f32) + b_ref[...]
        if relu:
            acc = jnp.maximum(acc, 0.0)
        o_ref[...] = acc
    N = x.shape[0]
    specs = [pl.BlockSpec((_BR, H), lambda i: (i, 0)),
             pl.BlockSpec((_BR, H), lambda i: (i, 0)),
             pl.BlockSpec((H, H), lambda i: (0, 0)),
             pl.BlockSpec((H, H), lambda i: (0, 0)),
             pl.BlockSpec((1, H), lambda i: (0, 0))]
    return _tc_call(body, N, H, 5, specs)(x, y, wx, wy, b)


def tc_mm_rowdot(x, w, v):
    def body(x_ref, w_ref, v_ref, h_ref, e_ref):
        hh = jnp.dot(x_ref[...], w_ref[...], preferred_element_type=f32)
        h_ref[...] = hh
        e_ref[...] = jnp.sum(hh * v_ref[...], axis=1, keepdims=True)
    N = x.shape[0]
    specs = [pl.BlockSpec((_BR, H), lambda i: (i, 0)),
             pl.BlockSpec((H, H), lambda i: (0, 0)),
             pl.BlockSpec((1, H), lambda i: (0, 0))]
    outs = (jax.ShapeDtypeStruct((N, H), f32),
            jax.ShapeDtypeStruct((N, 1), f32))
    return _tc_call(body, N, None, 3, specs, out_shapes=outs)(x, w, v)


def tc_mv(x, w, v):
    def body(x_ref, w_ref, v_ref, o_ref):
        hh = jnp.dot(x_ref[...], w_ref[...], preferred_element_type=f32)
        o_ref[...] = jnp.sum(hh * v_ref[...], axis=1, keepdims=True)
    N = x.shape[0]
    specs = [pl.BlockSpec((_BR, H), lambda i: (i, 0)),
             pl.BlockSpec((H, H), lambda i: (0, 0)),
             pl.BlockSpec((1, H), lambda i: (0, 0))]
    return _tc_call(body, N, 1, 3, specs)(x, w, v)


def tc_gcn_pre(h, d0, d1):
    def body(h_ref, d0_ref, d1_ref, hp_ref, di_ref):
        dinv = lax.rsqrt(d0_ref[...] + d1_ref[...] + 1.0)
        di_ref[...] = dinv
        hp_ref[...] = h_ref[...] * dinv
    N = h.shape[0]
    specs = [pl.BlockSpec((_BR, H), lambda i: (i, 0)),
             pl.BlockSpec((_BR, 1), lambda i: (i, 0)),
             pl.BlockSpec((_BR, 1), lambda i: (i, 0))]
    outs = (jax.ShapeDtypeStruct((N, H), f32),
            jax.ShapeDtypeStruct((N, 1), f32))
    return _tc_call(body, N, None, 3, specs, out_shapes=outs)(h, d0, d1)


def tc_gcn_fin(ga, gb, dinv, h, b):
    def body(ga_ref, gb_ref, di_ref, h_ref, b_ref, o_ref):
        di = di_ref[...]
        o_ref[...] = (ga_ref[...] + gb_ref[...]) * di \
            + h_ref[...] * (di * di) + b_ref[...]
    N = h.shape[0]
    specs = [pl.BlockSpec((_BR, H), lambda i: (i, 0)),
             pl.BlockSpec((_BR, H), lambda i: (i, 0)),
             pl.BlockSpec((_BR, 1), lambda i: (i, 0)),
             pl.BlockSpec((_BR, H), lambda i: (i, 0)),
             pl.BlockSpec((1, H), lambda i: (0, 0))]
    return _tc_call(body, N, H, 5, specs)(ga, gb, dinv, h, b)


def tc_gat_fin(oa, ob, b, relu):
    def body(oa_ref, ob_ref, b_ref, o_ref):
        acc = oa_ref[...] + ob_ref[...] + b_ref[...]
        if relu:
            acc = jnp.maximum(acc, 0.0)
        o_ref[...] = acc
    N = oa.shape[0]
    specs = [pl.BlockSpec((_BR, H), lambda i: (i, 0)),
             pl.BlockSpec((_BR, H), lambda i: (i, 0)),
             pl.BlockSpec((1, H), lambda i: (0, 0))]
    return _tc_call(body, N, H, 3, specs)(oa, ob, b)


def tc_winv(d0, d1):
    def body(d0_ref, d1_ref, o_ref):
        o_ref[...] = 1.0 / (d0_ref[...] + d1_ref[...] + 1e-16)
    N = d0.shape[0]
    specs = [pl.BlockSpec((_BR, 1), lambda i: (i, 0)),
             pl.BlockSpec((_BR, 1), lambda i: (i, 0))]
    return _tc_call(body, N, 1, 2, specs)(d0, d1)


def tc_scale_rows(rows, ex, wg):
    def body(r_ref, e_ref, w_ref, o_ref):
        o_ref[...] = r_ref[...] * (e_ref[...] * w_ref[...])
    N = rows.shape[0]
    specs = [pl.BlockSpec((_BR, H), lambda i: (i, 0)),
             pl.BlockSpec((_BR, 1), lambda i: (i, 0)),
             pl.BlockSpec((_BR, 1), lambda i: (i, 0))]
    return _tc_call(body, N, H, 3, specs)(rows, ex, wg)


def tc_sage_mean(sa, sb, c0, c1):
    def body(sa_ref, sb_ref, c0_ref, c1_ref, o_ref):
        cnt = jnp.maximum(c0_ref[...] + c1_ref[...], 1.0)
        o_ref[...] = (sa_ref[...] + sb_ref[...]) / cnt
    N = sa.shape[0]
    specs = [pl.BlockSpec((_BR, H), lambda i: (i, 0)),
             pl.BlockSpec((_BR, H), lambda i: (i, 0)),
             pl.BlockSpec((_BR, 1), lambda i: (i, 0)),
             pl.BlockSpec((_BR, 1), lambda i: (i, 0))]
    return _tc_call(body, N, H, 4, specs)(sa, sb, c0, c1)


def tc_combine_gene(gcn, a1, a2, b1, c1, c2, b2, sg):
    def body(g_ref, a1_ref, a2_ref, b1_ref, c1_ref, c2_ref, b2_ref, s_ref,
             o_ref):
        acc = g_ref[...] + a1_ref[...] + a2_ref[...] + b1_ref[...]
        acc = acc + c1_ref[...] + c2_ref[...] + b2_ref[...] + s_ref[...]
        o_ref[...] = jnp.maximum(acc, 0.0)
    N = gcn.shape[0]
    row = pl.BlockSpec((_BR, H), lambda i: (i, 0))
    bias = pl.BlockSpec((1, H), lambda i: (0, 0))
    specs = [row, row, row, bias, row, row, bias, row]
    return _tc_call(body, N, H, 8, specs)(gcn, a1, a2, b1, c1, c2, b2, sg)


def tc_dec(rs, rd, w1a, w1b, b1, w2row, b2):
    def body(rs_ref, rd_ref, wa_ref, wb_ref, b1_ref, w2_ref, b2_ref, o_ref):
        hh = jnp.dot(rs_ref[...], wa_ref[...], preferred_element_type=f32)
        hh = hh + jnp.dot(rd_ref[...], wb_ref[...], preferred_element_type=f32)
        hh = jnp.maximum(hh + b1_ref[...], 0.0)
        o_ref[...] = jnp.sum(hh * w2_ref[...], axis=1, keepdims=True) \
            + b2_ref[...]
    N = rs.shape[0]
    specs = [pl.BlockSpec((_BR, H), lambda i: (i, 0)),
             pl.BlockSpec((_BR, H), lambda i: (i, 0)),
             pl.BlockSpec((H, H), lambda i: (0, 0)),
             pl.BlockSpec((H, H), lambda i: (0, 0)),
             pl.BlockSpec((1, H), lambda i: (0, 0)),
             pl.BlockSpec((1, H), lambda i: (0, 0)),
             pl.BlockSpec((1, 1), lambda i: (0, 0))]
    return _tc_call(body, N, 1, 7, specs)(rs, rd, w1a, w1b, b1, w2row, b2)


def _zero_vec(ref, n):
    z = jnp.zeros((L,), f32)
    for k in range(n // L):
        ref[pl.ds(k * L, L)] = z


def sc_gather_rows(table, idx):
    Ep = idx.shape[0]
    ept = Ep // NW
    nwaves = ept // WAVE

    def body(table_h, idx_h, out_h, idx_v, rows_v, sem0, sem1):
        cid = lax.axis_index("c")
        sid = lax.axis_index("s")
        base = (sid * NC + cid) * ept
        sems = (sem0, sem1)

        def issue(w):
            p = w & 1
            off = base + w * WAVE
            pltpu.sync_copy(idx_h.at[pl.ds(off, WAVE)], idx_v.at[p])
            return pltpu.async_copy(table_h.at[idx_v.at[p]], rows_v.at[p],
                                    sems[p])

        h_prev = issue(0)
        for w in range(1, nwaves):
            h_cur = issue(w)
            h_prev.wait()
            pltpu.sync_copy(rows_v.at[(w - 1) & 1],
                            out_h.at[pl.ds(base + (w - 1) * WAVE, WAVE)])
            h_prev = h_cur
        h_prev.wait()
        pltpu.sync_copy(rows_v.at[(nwaves - 1) & 1],
                        out_h.at[pl.ds(base + (nwaves - 1) * WAVE, WAVE)])

    return pl.kernel(
        body,
        out_type=jax.ShapeDtypeStruct((Ep, H), f32),
        mesh=_mesh(),
        scratch_types=[pltpu.VMEM((2, WAVE), i32),
                       pltpu.VMEM((2, WAVE, H), f32),
                       pltpu.SemaphoreType.DMA,
                       pltpu.SemaphoreType.DMA],
    )(table, idx)


def sc_scatter_cols(table, src, dst, ndp, ncol):
    CW = H // ncol
    Np = table.shape[0]
    slabs = [table[:, c * CW:(c + 1) * CW] for c in range(ncol)]
    Ep = dst.shape[0]
    half = Ep // NC
    ept = half // NS
    nwaves = ept // WAVE
    stripe = ndp // NS
    assert stripe % 16 == 0
    BB = max(b for b in range(16, 129, 16) if stripe % b == 0)
    nbb = stripe // BB
    indirect = src is not None
    ins = tuple(slabs) + ((src,) if indirect else ()) + (dst,)

    def body(*refs):
        slab_hs = refs[:ncol]
        k = ncol
        if indirect:
            src_h = refs[k]
            k += 1
        dst_h = refs[k]
        out_h = refs[k + 1]
        sidx_v, loc2_v, rows_v, bounce_v, acc_sh, sem = refs[k + 2:]
        cid = lax.axis_index("c")
        sid = lax.axis_index("s")
        z16 = jnp.zeros((L,), f32)
        srow0 = sid * stripe
        tbase = cid * half + sid * ept
        if indirect:
            pltpu.sync_copy(src_h.at[pl.ds(tbase, ept)], sidx_v)
        def bw(w, _):
            pltpu.sync_copy(dst_h.at[pl.ds(tbase + w * WAVE, WAVE)],
                            loc2_v.at[w])
            for j in range(WAVE // L):
                d16 = loc2_v[w, pl.ds(j * L, L)]
                garb = ndp + (d16 & (GROWS - 1))
                loc2_v[w, pl.ds(j * L, L)] = jnp.where(d16 >= 0, d16, garb)
            return 0

        lax.fori_loop(0, nwaves, bw, 0)

        for c in range(ncol):
            for r in range(BB):
                for j in range(CW // L):
                    bounce_v[r, pl.ds(j * L, L)] = z16

            def zloop(r, _):
                pltpu.sync_copy(bounce_v,
                                acc_sh.at[pl.ds(srow0 + r * BB, BB)])
                return 0

            lax.fori_loop(0, nbb, zloop, 0)
            plsc.subcore_barrier()

            def wave(w, _):
                if indirect:
                    pltpu.async_copy(
                        slab_hs[c].at[sidx_v.at[pl.ds(w * WAVE, WAVE)]],
                        rows_v, sem).wait()
                else:
                    pltpu.sync_copy(
                        slab_hs[c].at[pl.ds(tbase + w * WAVE, WAVE)], rows_v)
                pltpu.sync_copy(rows_v, acc_sh.at[loc2_v.at[w]], add=True)
                return 0

            lax.fori_loop(0, nwaves, wave, 0)
            plsc.subcore_barrier()

            def oloop(r, _):
                pltpu.sync_copy(acc_sh.at[pl.ds(srow0 + r * BB, BB)],
                                bounce_v)
                pltpu.sync_copy(bounce_v,
                                out_h.at[cid, c,
                                         pl.ds(srow0 + r * BB, BB)])
                return 0

            lax.fori_loop(0, nbb, oloop, 0)
            plsc.subcore_barrier()

    scratch = [pltpu.VMEM((ept if indirect else L,), i32),
               pltpu.VMEM((nwaves, WAVE), i32),
               pltpu.VMEM((WAVE, CW), f32),
               pltpu.VMEM((BB, CW), f32),
               pltpu.VMEM_SHARED((ndp + GROWS, CW), f32),
               pltpu.SemaphoreType.DMA]
    out = pl.kernel(
        body,
        out_type=jax.ShapeDtypeStruct((NC, ncol, ndp, CW), f32),
        mesh=_mesh(),
        scratch_types=scratch,
        compiler_params=pltpu.CompilerParams(use_tc_tiling_on_sc=False),
    )(*ins)
    return out.transpose(0, 2, 1, 3).reshape(NC, ndp, H)


def _zero_spmem_1d(zvec_v, sh, s0, n):
    nbig, rem = divmod(n, 1024)
    for k in range(nbig):
        pltpu.sync_copy(zvec_v, sh.at[pl.ds(s0 + k * 1024, 1024)])
    if rem:
        pltpu.sync_copy(zvec_v.at[pl.ds(0, rem)],
                        sh.at[pl.ds(s0 + nbig * 1024, rem)])


def sc_gat_edge(es, ed, src, dst, ndp):
    Ep = src.shape[0]
    ept = Ep // NW
    nwaves = ept // WAVE
    stripe = ndp // NS

    def body(es_h, ed_h, src_h, dst_h, ex_h, den_h,
             sidx_v, didx_v, esg_v, edg_v, exw_v, zvec_v, dstripe_v,
             den_sh, sem):
        cid = lax.axis_index("c")
        sid = lax.axis_index("s")
        base = (sid * NC + cid) * ept
        _zero_vec(zvec_v, 1024)
        _zero_spmem_1d(zvec_v, den_sh, sid * stripe, stripe)
        plsc.subcore_barrier()

        def wave(w, _):
            off = base + w * WAVE
            pltpu.sync_copy(src_h.at[pl.ds(off, WAVE)], sidx_v)
            pltpu.sync_copy(dst_h.at[pl.ds(off, WAVE)], didx_v)
            pltpu.async_copy(es_h.at[sidx_v], esg_v, sem).wait()
            pltpu.async_copy(ed_h.at[didx_v], edg_v, sem).wait()
            for j in range(WAVE // L):
                ge = esg_v[pl.ds(j * L, L)] + edg_v[pl.ds(j * L, L)]
                ge = jnp.where(ge > 0, ge, NSLOPE * ge)
                exw_v[pl.ds(j * L, L)] = jnp.exp(ge)
            pltpu.sync_copy(exw_v, ex_h.at[pl.ds(off, WAVE)])
            pltpu.sync_copy(exw_v, den_sh.at[didx_v], add=True)
            return 0

        lax.fori_loop(0, nwaves, wave, 0)
        plsc.subcore_barrier()
        pltpu.sync_copy(den_sh.at[pl.ds(sid * stripe, stripe)], dstripe_v)
        pltpu.sync_copy(dstripe_v,
                        den_h.at[pl.ds(cid * ndp + sid * stripe, stripe)])

    return pl.kernel(
        body,
        out_type=(jax.ShapeDtypeStruct((Ep,), f32),
                  jax.ShapeDtypeStruct((NC * ndp,), f32)),
        mesh=_mesh(),
        scratch_types=[pltpu.VMEM((WAVE,), i32),
                       pltpu.VMEM((WAVE,), i32),
                       pltpu.VMEM((WAVE,), f32),
                       pltpu.VMEM((WAVE,), f32),
                       pltpu.VMEM((WAVE,), f32),
                       pltpu.VMEM((1024,), f32),
                       pltpu.VMEM((stripe,), f32),
                       pltpu.VMEM_SHARED((ndp,), f32),
                       pltpu.SemaphoreType.DMA],
    )(es, ed, src, dst)


def sc_gat_edge2(es, ed, src, dst, ndp):
    ex, den = sc_gat_edge(es, ed, src, dst, ndp)
    return ex, den.reshape(NC, ndp)


def sc_gather_scalar(vec, idx):
    Ep = idx.shape[0]
    ept = Ep // NW
    nwaves = ept // WAVE

    def body(vec_h, idx_h, out_h, idx_v, ob_v, sem):
        cid = lax.axis_index("c")
        sid = lax.axis_index("s")
        base = (sid * NC + cid) * ept

        def wave(w, _):
            off = base + w * WAVE
            pltpu.sync_copy(idx_h.at[pl.ds(off, WAVE)], idx_v)
            pltpu.async_copy(vec_h.at[idx_v], ob_v, sem).wait()
            pltpu.sync_copy(ob_v, out_h.at[pl.ds(off, WAVE)])
            return 0

        lax.fori_loop(0, nwaves, wave, 0)

    return pl.kernel(
        body,
        out_type=jax.ShapeDtypeStruct((Ep,), f32),
        mesh=_mesh(),
        scratch_types=[pltpu.VMEM((WAVE,), i32),
                       pltpu.VMEM((WAVE,), f32),
                       pltpu.SemaphoreType.DMA],
    )(vec, idx)


def sc_count(dst, ndp):
    Ep = dst.shape[0]
    ept = Ep // NW
    nwaves = ept // WAVE
    stripe = ndp // NS

    def body(dst_h, den_h, didx_v, ones_v, zvec_v, dstripe_v, den_sh):
        cid = lax.axis_index("c")
        sid = lax.axis_index("s")
        base = (sid * NC + cid) * ept
        one = jnp.ones((L,), f32)
        for k in range(WAVE // L):
            ones_v[pl.ds(k * L, L)] = one
        _zero_vec(zvec_v, 1024)
        _zero_spmem_1d(zvec_v, den_sh, sid * stripe, stripe)
        plsc.subcore_barrier()

        def wave(w, _):
            off = base + w * WAVE
            pltpu.sync_copy(dst_h.at[pl.ds(off, WAVE)], didx_v)
            pltpu.sync_copy(ones_v, den_sh.at[didx_v], add=True)
            return 0

        lax.fori_loop(0, nwaves, wave, 0)
        plsc.subcore_barrier()
        pltpu.sync_copy(den_sh.at[pl.ds(sid * stripe, stripe)], dstripe_v)
        pltpu.sync_copy(dstripe_v,
                        den_h.at[pl.ds(cid * ndp + sid * stripe, stripe)])

    return pl.kernel(
        body,
        out_type=jax.ShapeDtypeStruct((NC * ndp,), f32),
        mesh=_mesh(),
        scratch_types=[pltpu.VMEM((WAVE,), i32),
                       pltpu.VMEM((WAVE,), f32),
                       pltpu.VMEM((1024,), f32),
                       pltpu.VMEM((stripe,), f32),
                       pltpu.VMEM_SHARED((ndp,), f32)],
    )(dst).reshape(NC, ndp)


def _pad_rows(x, np_):
    return jnp.pad(x, ((0, np_ - x.shape[0]), (0, 0)))


def _pad_const(v, ep, fill):
    return jnp.pad(v, (0, ep - v.shape[0]), constant_values=fill)


def _pad_spread(v, ep, base, span):
    npad = ep - v.shape[0]
    tail = base + (jnp.arange(npad, dtype=i32) % span)
    return jnp.concatenate([v, tail])


def _edges_scatter(ei, ep):
    return (_pad_const(ei[0], ep, 0),
            _pad_spread(ei[1], ep, -GROWS, GROWS))


def _mask_tail(e_col, n):
    idx = jnp.arange(e_col.shape[0], dtype=i32)[:, None]
    return jnp.where(idx < n, e_col, -1e30)


def _gcn(z, ei_pack, p, ngp):
    src0, dstn, dstc = ei_pack
    h = tc_mm(z, p["W"], jnp.zeros((1, H), f32))
    degp = sc_count(dstc, ngp)
    hpre, dinv = tc_gcn_pre(h, degp[0][:, None], degp[1][:, None])
    gp = sc_scatter_cols(hpre, src0, dstn, ngp, 4)
    return tc_gcn_fin(gp[0], gp[1], dinv, h, p["b"].reshape(1, H))


def _gat(zs, zd, pack, p, ns, nsp, ndp, ncol):
    src0, srcs, dstd, dstn = pack
    hs, es = tc_mm_rowdot(zs, p["Ws"], p["as_"].reshape(1, H))
    ed = tc_mv(zd, p["Wd"], p["ad"].reshape(1, H))
    es_m = _mask_tail(es, ns).reshape(-1)
    exv, denp = sc_gat_edge2(es_m, ed.reshape(-1), srcs, dstd, ndp)
    winv = tc_winv(denp[0][:, None], denp[1][:, None])
    wg = sc_gather_scalar(winv.reshape(-1), dstd)
    rows = sc_gather_rows(hs, src0)
    msg = tc_scale_rows(rows, exv[:, None], wg[:, None])
    op_ = sc_scatter_cols(msg, None, dstn, ndp, ncol)
    return op_[0], op_[1]


def _sage(zs, zd, pack, p, ndp, ncol, relu):
    src0, dstn, dstc = pack
    sp = sc_scatter_cols(zs, src0, dstn, ndp, ncol)
    cp = sc_count(dstc, ndp)
    mean = tc_sage_mean(sp[0], sp[1], cp[0][:, None], cp[1][:, None])
    return tc_mm2(mean, zd, p["Wl"], p["Wr"], p["bl"].reshape(1, H),
                  relu=relu)


def kernel(x_gene, x_msig, x_reactome, x_bp,
           ei_g2g, ei_genemsig, ei_genereact, ei_genebp,
           ei_rev_genemsig, ei_rev_genereact, ei_rev_genebp,
           el_gene_gene, el_gene_msig, el_gene_reactome, el_gene_bp,
           el_msig_gene, el_reactome_gene, el_bp_gene, params):
    NG, NM, NR, NB = (x_gene.shape[0], x_msig.shape[0],
                      x_reactome.shape[0], x_bp.shape[0])
    NGp, NMp, NRp, NBp = (_rup(NG, 256), _rup(NM, 256),
                          _rup(NR, 256), _rup(NB, 512))
    sizes = {"gene": (NG, NGp), "msig": (NM, NMp),
             "reactome": (NR, NRp), "bp": (NB, NBp)}

    z = {"gene": _pad_rows(x_gene, NGp), "msig": _pad_rows(x_msig, NMp),
         "reactome": _pad_rows(x_reactome, NRp), "bp": _pad_rows(x_bp, NBp)}

    eg = _rup(ei_g2g.shape[1], EALIGN)
    er = _rup(ei_genemsig.shape[1], EALIGN)
    elp = _rup(el_gene_gene.shape[1], EALIGN)

    g2g_pack = _edges_scatter(ei_g2g, eg) + (
        _pad_spread(ei_g2g[1], eg, NG, NGp - NG),)

    def gat_pack(ei, ns, nsp, nd, ndp):
        return (_pad_const(ei[0], er, 0),
                _pad_const(ei[0], er, ns),
                _pad_spread(ei[1], er, nd, ndp - nd),
                _pad_spread(ei[1], er, -GROWS, GROWS))

    def sage_pack(ei, nd, ndp):
        return _edges_scatter(ei, er) + (
            _pad_spread(ei[1], er, nd, ndp - nd),)

    packs = {
        "rev_genemsig": gat_pack(ei_rev_genemsig, NM, NMp, NG, NGp),
        "rev_genereact": gat_pack(ei_rev_genereact, NR, NRp, NG, NGp),
        "rev_genebp": sage_pack(ei_rev_genebp, NG, NGp),
        "genemsig": gat_pack(ei_genemsig, NG, NGp, NM, NMp),
        "genereact": gat_pack(ei_genereact, NG, NGp, NR, NRp),
        "genebp": sage_pack(ei_genebp, NB, NBp),
    }

    for p in params["layers"]:
        gcn = _gcn(z["gene"], g2g_pack, p["g2g"], NGp)
        a1, a2 = _gat(z["msig"], z["gene"], packs["rev_genemsig"],
                      p["rev_genemsig"], NM, NMp, NGp, 4)
        c1, c2 = _gat(z["reactome"], z["gene"], packs["rev_genereact"],
                      p["rev_genereact"], NR, NRp, NGp, 4)
        sg = _sage(z["bp"], z["gene"], packs["rev_genebp"], p["rev_genebp"],
                   NGp, 4, relu=False)
        m1, m2 = _gat(z["gene"], z["msig"], packs["genemsig"], p["genemsig"],
                      NG, NGp, NMp, 1)
        r1, r2 = _gat(z["gene"], z["reactome"], packs["genereact"],
                      p["genereact"], NG, NGp, NRp, 1)
        bpo = _sage(z["gene"], z["bp"], packs["genebp"], p["genebp"], NBp, 2,
                    relu=True)
        gene_new = tc_combine_gene(
            gcn, a1, a2, p["rev_genemsig"]["b"].reshape(1, H),
            c1, c2, p["rev_genereact"]["b"].reshape(1, H), sg)
        z = {"gene": gene_new,
             "msig": tc_gat_fin(m1, m2, p["genemsig"]["b"].reshape(1, H),
                                relu=True),
             "reactome": tc_gat_fin(r1, r2,
                                    p["genereact"]["b"].reshape(1, H),
                                    relu=True),
             "bp": bpo}

    rel_keys = [("gene_gene", "gene", "gene"), ("gene_msig", "gene", "msig"),
                ("gene_reactome", "gene", "reactome"),
                ("gene_bp", "gene", "bp"), ("msig_gene", "msig", "gene"),
                ("reactome_gene", "reactome", "gene"),
                ("bp_gene", "bp", "gene")]
    els = {"gene_gene": el_gene_gene, "gene_msig": el_gene_msig,
           "gene_reactome": el_gene_reactome, "gene_bp": el_gene_bp,
           "msig_gene": el_msig_gene, "reactome_gene": el_reactome_gene,
           "bp_gene": el_bp_gene}
    E_LBL = el_gene_gene.shape[1]
    scores = []
    for key, st, dt in rel_keys:
        el = els[key]
        pd = params["dec"][key]
        rs = sc_gather_rows(z[st], _pad_const(el[0], elp, 0))
        rd = sc_gather_rows(z[dt], _pad_const(el[1], elp, 0))
        sc = tc_dec(rs, rd, pd["W1"][:H], pd["W1"][H:],
                    pd["b1"].reshape(1, H), pd["W2"].reshape(1, H),
                    pd["b2"].reshape(1, 1))
        scores.append(sc[:E_LBL])

    return (z["gene"][:NG], z["msig"][:NM], z["reactome"][:NR],
            z["bp"][:NB]) + tuple(scores)


# speedup vs baseline: 1.0304x; 1.0304x over previous
import functools

import jax
import jax.numpy as jnp
from jax import lax
from jax.experimental import pallas as pl
from jax.experimental.pallas import tpu as pltpu
from jax.experimental.pallas import tpu_sc as plsc

H = 128
NC, NS, L = 2, 16, 16
NW = NC * NS
WAVE = 128
EALIGN = NW * WAVE
GROWS = 256
NSLOPE = 0.2

f32 = jnp.float32
i32 = jnp.int32


def _rup(n, m):
    return ((n + m - 1) // m) * m


def _mesh():
    return plsc.VectorSubcoreMesh(core_axis_name="c", subcore_axis_name="s",
                                  num_cores=NC, num_subcores=NS)


_BR = 256


def _tc_call(body, nrows, out_width, n_in, extra_specs, out_shapes=None):
    grid = (nrows // _BR,)
    if out_shapes is None:
        out_shapes = jax.ShapeDtypeStruct((nrows, out_width), f32)
        out_specs = pl.BlockSpec((_BR, out_width), lambda i: (i, 0))
    else:
        out_specs = [pl.BlockSpec((_BR, s.shape[1]), lambda i: (i, 0))
                     for s in out_shapes]
    return pl.pallas_call(
        body, grid=grid,
        in_specs=extra_specs,
        out_specs=out_specs,
        out_shape=out_shapes,
    )


def tc_mm(x, w, b):
    def body(x_ref, w_ref, b_ref, o_ref):
        o_ref[...] = jnp.dot(x_ref[...], w_ref[...],
                             preferred_element_type=f32) + b_ref[...]
    N = x.shape[0]
    specs = [pl.BlockSpec((_BR, H), lambda i: (i, 0)),
             pl.BlockSpec((H, H), lambda i: (0, 0)),
             pl.BlockSpec((1, H), lambda i: (0, 0))]
    return _tc_call(body, N, H, 3, specs)(x, w, b)


def tc_mm2(x, y, wx, wy, b, relu):
    def body(x_ref, y_ref, wx_ref, wy_ref, b_ref, o_ref):
        acc = jnp.dot(x_ref[...], wx_ref[...], preferred_element_type=f32)
        acc = acc + jnp.dot(y_ref[...], wy_ref[...],
                            preferred_element_type=f32) + b_ref[...]
        if relu:
            acc = jnp.maximum(acc, 0.0)
        o_ref[...] = acc
    N = x.shape[0]
    specs = [pl.BlockSpec((_BR, H), lambda i: (i, 0)),
             pl.BlockSpec((_BR, H), lambda i: (i, 0)),
             pl.BlockSpec((H, H), lambda i: (0, 0)),
             pl.BlockSpec((H, H), lambda i: (0, 0)),
             pl.BlockSpec((1, H), lambda i: (0, 0))]
    return _tc_call(body, N, H, 5, specs)(x, y, wx, wy, b)


def tc_mm_rowdot(x, w, v):
    def body(x_ref, w_ref, v_ref, h_ref, e_ref):
        hh = jnp.dot(x_ref[...], w_ref[...], preferred_element_type=f32)
        h_ref[...] = hh
        e_ref[...] = jnp.sum(hh * v_ref[...], axis=1, keepdims=True)
    N = x.shape[0]
    specs = [pl.BlockSpec((_BR, H), lambda i: (i, 0)),
             pl.BlockSpec((H, H), lambda i: (0, 0)),
             pl.BlockSpec((1, H), lambda i: (0, 0))]
    outs = (jax.ShapeDtypeStruct((N, H), f32),
            jax.ShapeDtypeStruct((N, 1), f32))
    return _tc_call(body, N, None, 3, specs, out_shapes=outs)(x, w, v)


def tc_mv(x, w, v):
    def body(x_ref, w_ref, v_ref, o_ref):
        hh = jnp.dot(x_ref[...], w_ref[...], preferred_element_type=f32)
        o_ref[...] = jnp.sum(hh * v_ref[...], axis=1, keepdims=True)
    N = x.shape[0]
    specs = [pl.BlockSpec((_BR, H), lambda i: (i, 0)),
             pl.BlockSpec((H, H), lambda i: (0, 0)),
             pl.BlockSpec((1, H), lambda i: (0, 0))]
    return _tc_call(body, N, 1, 3, specs)(x, w, v)


def tc_gcn_pre(h, d0, d1):
    def body(h_ref, d0_ref, d1_ref, hp_ref, di_ref):
        dinv = lax.rsqrt(d0_ref[...] + d1_ref[...] + 1.0)
        di_ref[...] = dinv
        hp_ref[...] = h_ref[...] * dinv
    N = h.shape[0]
    specs = [pl.BlockSpec((_BR, H), lambda i: (i, 0)),
             pl.BlockSpec((_BR, 1), lambda i: (i, 0)),
             pl.BlockSpec((_BR, 1), lambda i: (i, 0))]
    outs = (jax.ShapeDtypeStruct((N, H), f32),
            jax.ShapeDtypeStruct((N, 1), f32))
    return _tc_call(body, N, None, 3, specs, out_shapes=outs)(h, d0, d1)


def tc_gcn_fin(ga, gb, dinv, h, b):
    def body(ga_ref, gb_ref, di_ref, h_ref, b_ref, o_ref):
        di = di_ref[...]
        o_ref[...] = (ga_ref[...] + gb_ref[...]) * di \
            + h_ref[...] * (di * di) + b_ref[...]
    N = h.shape[0]
    specs = [pl.BlockSpec((_BR, H), lambda i: (i, 0)),
             pl.BlockSpec((_BR, H), lambda i: (i, 0)),
             pl.BlockSpec((_BR, 1), lambda i: (i, 0)),
             pl.BlockSpec((_BR, H), lambda i: (i, 0)),
             pl.BlockSpec((1, H), lambda i: (0, 0))]
    return _tc_call(body, N, H, 5, specs)(ga, gb, dinv, h, b)


def tc_gat_fin(oa, ob, winv, b, relu):
    def body(oa_ref, ob_ref, w_ref, b_ref, o_ref):
        acc = (oa_ref[...] + ob_ref[...]) * w_ref[...] + b_ref[...]
        if relu:
            acc = jnp.maximum(acc, 0.0)
        o_ref[...] = acc
    N = oa.shape[0]
    specs = [pl.BlockSpec((_BR, H), lambda i: (i, 0)),
             pl.BlockSpec((_BR, H), lambda i: (i, 0)),
             pl.BlockSpec((_BR, 1), lambda i: (i, 0)),
             pl.BlockSpec((1, H), lambda i: (0, 0))]
    return _tc_call(body, N, H, 4, specs)(oa, ob, winv, b)


def tc_winv(d0, d1):
    def body(d0_ref, d1_ref, o_ref):
        o_ref[...] = 1.0 / (d0_ref[...] + d1_ref[...] + 1e-16)
    N = d0.shape[0]
    specs = [pl.BlockSpec((_BR, 1), lambda i: (i, 0)),
             pl.BlockSpec((_BR, 1), lambda i: (i, 0))]
    return _tc_call(body, N, 1, 2, specs)(d0, d1)


def tc_scale_rows(rows, ex):
    def body(r_ref, e_ref, o_ref):
        o_ref[...] = r_ref[...] * e_ref[...]
    N = rows.shape[0]
    specs = [pl.BlockSpec((_BR, H), lambda i: (i, 0)),
             pl.BlockSpec((_BR, 1), lambda i: (i, 0))]
    return _tc_call(body, N, H, 2, specs)(rows, ex)


def tc_sage_mean(sa, sb, c0, c1):
    def body(sa_ref, sb_ref, c0_ref, c1_ref, o_ref):
        cnt = jnp.maximum(c0_ref[...] + c1_ref[...], 1.0)
        o_ref[...] = (sa_ref[...] + sb_ref[...]) / cnt
    N = sa.shape[0]
    specs = [pl.BlockSpec((_BR, H), lambda i: (i, 0)),
             pl.BlockSpec((_BR, H), lambda i: (i, 0)),
             pl.BlockSpec((_BR, 1), lambda i: (i, 0)),
             pl.BlockSpec((_BR, 1), lambda i: (i, 0))]
    return _tc_call(body, N, H, 4, specs)(sa, sb, c0, c1)


def tc_combine_gene(gcn, a1, a2, wa, b1, c1, c2, wc, b2, sg):
    def body(g_ref, a1_ref, a2_ref, wa_ref, b1_ref, c1_ref, c2_ref, wc_ref,
             b2_ref, s_ref, o_ref):
        acc = g_ref[...] + (a1_ref[...] + a2_ref[...]) * wa_ref[...] \
            + b1_ref[...]
        acc = acc + (c1_ref[...] + c2_ref[...]) * wc_ref[...] + b2_ref[...] \
            + s_ref[...]
        o_ref[...] = jnp.maximum(acc, 0.0)
    N = gcn.shape[0]
    row = pl.BlockSpec((_BR, H), lambda i: (i, 0))
    col = pl.BlockSpec((_BR, 1), lambda i: (i, 0))
    bias = pl.BlockSpec((1, H), lambda i: (0, 0))
    specs = [row, row, row, col, bias, row, row, col, bias, row]
    return _tc_call(body, N, H, 10, specs)(gcn, a1, a2, wa, b1, c1, c2, wc,
                                           b2, sg)


def tc_dec(rs, rd, w1a, w1b, b1, w2row, b2):
    def body(rs_ref, rd_ref, wa_ref, wb_ref, b1_ref, w2_ref, b2_ref, o_ref):
        hh = jnp.dot(rs_ref[...], wa_ref[...], preferred_element_type=f32)
        hh = hh + jnp.dot(rd_ref[...], wb_ref[...], preferred_element_type=f32)
        hh = jnp.maximum(hh + b1_ref[...], 0.0)
        o_ref[...] = jnp.sum(hh * w2_ref[...], axis=1, keepdims=True) \
            + b2_ref[...]
    N = rs.shape[0]
    specs = [pl.BlockSpec((_BR, H), lambda i: (i, 0)),
             pl.BlockSpec((_BR, H), lambda i: (i, 0)),
             pl.BlockSpec((H, H), lambda i: (0, 0)),
             pl.BlockSpec((H, H), lambda i: (0, 0)),
             pl.BlockSpec((1, H), lambda i: (0, 0)),
             pl.BlockSpec((1, H), lambda i: (0, 0)),
             pl.BlockSpec((1, 1), lambda i: (0, 0))]
    return _tc_call(body, N, 1, 7, specs)(rs, rd, w1a, w1b, b1, w2row, b2)


def _zero_vec(ref, n):
    z = jnp.zeros((L,), f32)
    for k in range(n // L):
        ref[pl.ds(k * L, L)] = z


def sc_gather_rows(table, idx):
    Ep = idx.shape[0]
    ept = Ep // NW
    nwaves = ept // WAVE

    def body(table_h, idx_h, out_h, idx_v, rows_v, sem0, sem1):
        cid = lax.axis_index("c")
        sid = lax.axis_index("s")
        base = (sid * NC + cid) * ept
        sems = (sem0, sem1)

        def issue(w):
            p = w & 1
            off = base + w * WAVE
            pltpu.sync_copy(idx_h.at[pl.ds(off, WAVE)], idx_v.at[p])
            return pltpu.async_copy(table_h.at[idx_v.at[p]], rows_v.at[p],
                                    sems[p])

        h_prev = issue(0)
        for w in range(1, nwaves):
            h_cur = issue(w)
            h_prev.wait()
            pltpu.sync_copy(rows_v.at[(w - 1) & 1],
                            out_h.at[pl.ds(base + (w - 1) * WAVE, WAVE)])
            h_prev = h_cur
        h_prev.wait()
        pltpu.sync_copy(rows_v.at[(nwaves - 1) & 1],
                        out_h.at[pl.ds(base + (nwaves - 1) * WAVE, WAVE)])

    return pl.kernel(
        body,
        out_type=jax.ShapeDtypeStruct((Ep, H), f32),
        mesh=_mesh(),
        scratch_types=[pltpu.VMEM((2, WAVE), i32),
                       pltpu.VMEM((2, WAVE, H), f32),
                       pltpu.SemaphoreType.DMA,
                       pltpu.SemaphoreType.DMA],
    )(table, idx)


def sc_scatter_cols(table, src, dst, ndp, ncol):
    CW = H // ncol
    Np = table.shape[0]
    slabs = [table[:, c * CW:(c + 1) * CW] for c in range(ncol)]
    Ep = dst.shape[0]
    half = Ep // NC
    ept = half // NS
    nwaves = ept // WAVE
    stripe = ndp // NS
    assert stripe % 16 == 0
    BB = max(b for b in range(16, 129, 16) if stripe % b == 0)
    nbb = stripe // BB
    indirect = src is not None
    ins = tuple(slabs) + ((src,) if indirect else ()) + (dst,)

    def body(*refs):
        slab_hs = refs[:ncol]
        k = ncol
        if indirect:
            src_h = refs[k]
            k += 1
        dst_h = refs[k]
        out_h = refs[k + 1]
        sidx_v, loc2_v, rows_v, bounce_v, acc_sh, sem = refs[k + 2:]
        cid = lax.axis_index("c")
        sid = lax.axis_index("s")
        z16 = jnp.zeros((L,), f32)
        srow0 = sid * stripe
        tbase = cid * half + sid * ept
        if indirect:
            pltpu.sync_copy(src_h.at[pl.ds(tbase, ept)], sidx_v)
        def bw(w, _):
            pltpu.sync_copy(dst_h.at[pl.ds(tbase + w * WAVE, WAVE)],
                            loc2_v.at[w])
            for j in range(WAVE // L):
                d16 = loc2_v[w, pl.ds(j * L, L)]
                garb = ndp + (d16 & (GROWS - 1))
                loc2_v[w, pl.ds(j * L, L)] = jnp.where(d16 >= 0, d16, garb)
            return 0

        lax.fori_loop(0, nwaves, bw, 0)

        for c in range(ncol):
            for r in range(BB):
                for j in range(CW // L):
                    bounce_v[r, pl.ds(j * L, L)] = z16

            def zloop(r, _):
                pltpu.sync_copy(bounce_v,
                                acc_sh.at[pl.ds(srow0 + r * BB, BB)])
                return 0

            lax.fori_loop(0, nbb, zloop, 0)
            plsc.subcore_barrier()

            def wave(w, _):
                if indirect:
                    pltpu.async_copy(
                        slab_hs[c].at[sidx_v.at[pl.ds(w * WAVE, WAVE)]],
                        rows_v, sem).wait()
                else:
                    pltpu.sync_copy(
                        slab_hs[c].at[pl.ds(tbase + w * WAVE, WAVE)], rows_v)
                pltpu.sync_copy(rows_v, acc_sh.at[loc2_v.at[w]], add=True)
                return 0

            lax.fori_loop(0, nwaves, wave, 0)
            plsc.subcore_barrier()

            def oloop(r, _):
                pltpu.sync_copy(acc_sh.at[pl.ds(srow0 + r * BB, BB)],
                                bounce_v)
                pltpu.sync_copy(bounce_v,
                                out_h.at[cid, c,
                                         pl.ds(srow0 + r * BB, BB)])
                return 0

            lax.fori_loop(0, nbb, oloop, 0)
            plsc.subcore_barrier()

    scratch = [pltpu.VMEM((ept if indirect else L,), i32),
               pltpu.VMEM((nwaves, WAVE), i32),
               pltpu.VMEM((WAVE, CW), f32),
               pltpu.VMEM((BB, CW), f32),
               pltpu.VMEM_SHARED((ndp + GROWS, CW), f32),
               pltpu.SemaphoreType.DMA]
    out = pl.kernel(
        body,
        out_type=jax.ShapeDtypeStruct((NC, ncol, ndp, CW), f32),
        mesh=_mesh(),
        scratch_types=scratch,
        compiler_params=pltpu.CompilerParams(use_tc_tiling_on_sc=False),
    )(*ins)
    return out.transpose(0, 2, 1, 3).reshape(NC, ndp, H)


def _zero_spmem_1d(zvec_v, sh, s0, n):
    nbig, rem = divmod(n, 1024)
    for k in range(nbig):
        pltpu.sync_copy(zvec_v, sh.at[pl.ds(s0 + k * 1024, 1024)])
    if rem:
        pltpu.sync_copy(zvec_v.at[pl.ds(0, rem)],
                        sh.at[pl.ds(s0 + nbig * 1024, rem)])


def sc_gat_edge(es, ed, src, dst, ndp):
    Ep = src.shape[0]
    ept = Ep // NW
    nwaves = ept // WAVE
    stripe = ndp // NS

    def body(es_h, ed_h, src_h, dst_h, ex_h, den_h,
             sidx_v, didx_v, esg_v, edg_v, exw_v, zvec_v, dstripe_v,
             den_sh, sem):
        cid = lax.axis_index("c")
        sid = lax.axis_index("s")
        base = (sid * NC + cid) * ept
        _zero_vec(zvec_v, 1024)
        _zero_spmem_1d(zvec_v, den_sh, sid * stripe, stripe)
        plsc.subcore_barrier()

        def wave(w, _):
            off = base + w * WAVE
            pltpu.sync_copy(src_h.at[pl.ds(off, WAVE)], sidx_v)
            pltpu.sync_copy(dst_h.at[pl.ds(off, WAVE)], didx_v)
            pltpu.async_copy(es_h.at[sidx_v], esg_v, sem).wait()
            pltpu.async_copy(ed_h.at[didx_v], edg_v, sem).wait()
            for j in range(WAVE // L):
                ge = esg_v[pl.ds(j * L, L)] + edg_v[pl.ds(j * L, L)]
                ge = jnp.where(ge > 0, ge, NSLOPE * ge)
                exw_v[pl.ds(j * L, L)] = jnp.exp(ge)
            pltpu.sync_copy(exw_v, ex_h.at[pl.ds(off, WAVE)])
            pltpu.sync_copy(exw_v, den_sh.at[didx_v], add=True)
            return 0

        lax.fori_loop(0, nwaves, wave, 0)
        plsc.subcore_barrier()
        pltpu.sync_copy(den_sh.at[pl.ds(sid * stripe, stripe)], dstripe_v)
        pltpu.sync_copy(dstripe_v,
                        den_h.at[pl.ds(cid * ndp + sid * stripe, stripe)])

    return pl.kernel(
        body,
        out_type=(jax.ShapeDtypeStruct((Ep,), f32),
                  jax.ShapeDtypeStruct((NC * ndp,), f32)),
        mesh=_mesh(),
        scratch_types=[pltpu.VMEM((WAVE,), i32),
                       pltpu.VMEM((WAVE,), i32),
                       pltpu.VMEM((WAVE,), f32),
                       pltpu.VMEM((WAVE,), f32),
                       pltpu.VMEM((WAVE,), f32),
                       pltpu.VMEM((1024,), f32),
                       pltpu.VMEM((stripe,), f32),
                       pltpu.VMEM_SHARED((ndp,), f32),
                       pltpu.SemaphoreType.DMA],
    )(es, ed, src, dst)


def sc_gat_edge2(es, ed, src, dst, ndp):
    ex, den = sc_gat_edge(es, ed, src, dst, ndp)
    return ex, den.reshape(NC, ndp)


def sc_gather_scalar(vec, idx):
    Ep = idx.shape[0]
    ept = Ep // NW
    nwaves = ept // WAVE

    def body(vec_h, idx_h, out_h, idx_v, ob_v, sem):
        cid = lax.axis_index("c")
        sid = lax.axis_index("s")
        base = (sid * NC + cid) * ept

        def wave(w, _):
            off = base + w * WAVE
            pltpu.sync_copy(idx_h.at[pl.ds(off, WAVE)], idx_v)
            pltpu.async_copy(vec_h.at[idx_v], ob_v, sem).wait()
            pltpu.sync_copy(ob_v, out_h.at[pl.ds(off, WAVE)])
            return 0

        lax.fori_loop(0, nwaves, wave, 0)

    return pl.kernel(
        body,
        out_type=jax.ShapeDtypeStruct((Ep,), f32),
        mesh=_mesh(),
        scratch_types=[pltpu.VMEM((WAVE,), i32),
                       pltpu.VMEM((WAVE,), f32),
                       pltpu.SemaphoreType.DMA],
    )(vec, idx)


def sc_count(dst, ndp):
    Ep = dst.shape[0]
    ept = Ep // NW
    nwaves = ept // WAVE
    stripe = ndp // NS

    def body(dst_h, den_h, didx_v, ones_v, zvec_v, dstripe_v, den_sh):
        cid = lax.axis_index("c")
        sid = lax.axis_index("s")
        base = (sid * NC + cid) * ept
        one = jnp.ones((L,), f32)
        for k in range(WAVE // L):
            ones_v[pl.ds(k * L, L)] = one
        _zero_vec(zvec_v, 1024)
        _zero_spmem_1d(zvec_v, den_sh, sid * stripe, stripe)
        plsc.subcore_barrier()

        def wave(w, _):
            off = base + w * WAVE
            pltpu.sync_copy(dst_h.at[pl.ds(off, WAVE)], didx_v)
            pltpu.sync_copy(ones_v, den_sh.at[didx_v], add=True)
            return 0

        lax.fori_loop(0, nwaves, wave, 0)
        plsc.subcore_barrier()
        pltpu.sync_copy(den_sh.at[pl.ds(sid * stripe, stripe)], dstripe_v)
        pltpu.sync_copy(dstripe_v,
                        den_h.at[pl.ds(cid * ndp + sid * stripe, stripe)])

    return pl.kernel(
        body,
        out_type=jax.ShapeDtypeStruct((NC * ndp,), f32),
        mesh=_mesh(),
        scratch_types=[pltpu.VMEM((WAVE,), i32),
                       pltpu.VMEM((WAVE,), f32),
                       pltpu.VMEM((1024,), f32),
                       pltpu.VMEM((stripe,), f32),
                       pltpu.VMEM_SHARED((ndp,), f32)],
    )(dst).reshape(NC, ndp)


def _pad_rows(x, np_):
    return jnp.pad(x, ((0, np_ - x.shape[0]), (0, 0)))


def _pad_const(v, ep, fill):
    return jnp.pad(v, (0, ep - v.shape[0]), constant_values=fill)


def _pad_spread(v, ep, base, span):
    npad = ep - v.shape[0]
    tail = base + (jnp.arange(npad, dtype=i32) % span)
    return jnp.concatenate([v, tail])


def _edges_scatter(ei, ep):
    return (_pad_const(ei[0], ep, 0),
            _pad_spread(ei[1], ep, -GROWS, GROWS))


def _mask_tail(e_col, n):
    idx = jnp.arange(e_col.shape[0], dtype=i32)[:, None]
    return jnp.where(idx < n, e_col, -1e30)


def _gcn(z, ei_pack, p, ngp):
    src0, dstn, dstc = ei_pack
    h = tc_mm(z, p["W"], jnp.zeros((1, H), f32))
    degp = sc_count(dstc, ngp)
    hpre, dinv = tc_gcn_pre(h, degp[0][:, None], degp[1][:, None])
    gp = sc_scatter_cols(hpre, src0, dstn, ngp, 4)
    return tc_gcn_fin(gp[0], gp[1], dinv, h, p["b"].reshape(1, H))


def _gat(zs, zd, pack, p, ns, nsp, ndp, ncol):
    src0, srcs, dstd, dstn = pack
    hs, es = tc_mm_rowdot(zs, p["Ws"], p["as_"].reshape(1, H))
    ed = tc_mv(zd, p["Wd"], p["ad"].reshape(1, H))
    es_m = _mask_tail(es, ns).reshape(-1)
    exv, denp = sc_gat_edge2(es_m, ed.reshape(-1), srcs, dstd, ndp)
    winv = tc_winv(denp[0][:, None], denp[1][:, None])
    rows = sc_gather_rows(hs, src0)
    msg = tc_scale_rows(rows, exv[:, None])
    op_ = sc_scatter_cols(msg, None, dstn, ndp, ncol)
    return op_[0], op_[1], winv


def _sage(zs, zd, pack, p, ndp, ncol, relu):
    src0, dstn, dstc = pack
    sp = sc_scatter_cols(zs, src0, dstn, ndp, ncol)
    cp = sc_count(dstc, ndp)
    mean = tc_sage_mean(sp[0], sp[1], cp[0][:, None], cp[1][:, None])
    return tc_mm2(mean, zd, p["Wl"], p["Wr"], p["bl"].reshape(1, H),
                  relu=relu)


def kernel(x_gene, x_msig, x_reactome, x_bp,
           ei_g2g, ei_genemsig, ei_genereact, ei_genebp,
           ei_rev_genemsig, ei_rev_genereact, ei_rev_genebp,
           el_gene_gene, el_gene_msig, el_gene_reactome, el_gene_bp,
           el_msig_gene, el_reactome_gene, el_bp_gene, params):
    NG, NM, NR, NB = (x_gene.shape[0], x_msig.shape[0],
                      x_reactome.shape[0], x_bp.shape[0])
    NGp, NMp, NRp, NBp = (_rup(NG, 256), _rup(NM, 256),
                          _rup(NR, 256), _rup(NB, 512))
    sizes = {"gene": (NG, NGp), "msig": (NM, NMp),
             "reactome": (NR, NRp), "bp": (NB, NBp)}

    z = {"gene": _pad_rows(x_gene, NGp), "msig": _pad_rows(x_msig, NMp),
         "reactome": _pad_rows(x_reactome, NRp), "bp": _pad_rows(x_bp, NBp)}

    eg = _rup(ei_g2g.shape[1], EALIGN)
    er = _rup(ei_genemsig.shape[1], EALIGN)
    elp = _rup(el_gene_gene.shape[1], EALIGN)

    g2g_pack = _edges_scatter(ei_g2g, eg) + (
        _pad_spread(ei_g2g[1], eg, NG, NGp - NG),)

    def gat_pack(ei, ns, nsp, nd, ndp):
        return (_pad_const(ei[0], er, 0),
                _pad_const(ei[0], er, ns),
                _pad_spread(ei[1], er, nd, ndp - nd),
                _pad_spread(ei[1], er, -GROWS, GROWS))

    def sage_pack(ei, nd, ndp):
        return _edges_scatter(ei, er) + (
            _pad_spread(ei[1], er, nd, ndp - nd),)

    packs = {
        "rev_genemsig": gat_pack(ei_rev_genemsig, NM, NMp, NG, NGp),
        "rev_genereact": gat_pack(ei_rev_genereact, NR, NRp, NG, NGp),
        "rev_genebp": sage_pack(ei_rev_genebp, NG, NGp),
        "genemsig": gat_pack(ei_genemsig, NG, NGp, NM, NMp),
        "genereact": gat_pack(ei_genereact, NG, NGp, NR, NRp),
        "genebp": sage_pack(ei_genebp, NB, NBp),
    }

    for p in params["layers"]:
        gcn = _gcn(z["gene"], g2g_pack, p["g2g"], NGp)
        a1, a2, wa = _gat(z["msig"], z["gene"], packs["rev_genemsig"],
                          p["rev_genemsig"], NM, NMp, NGp, 4)
        c1, c2, wc = _gat(z["reactome"], z["gene"], packs["rev_genereact"],
                          p["rev_genereact"], NR, NRp, NGp, 4)
        sg = _sage(z["bp"], z["gene"], packs["rev_genebp"], p["rev_genebp"],
                   NGp, 4, relu=False)
        m1, m2, wm = _gat(z["gene"], z["msig"], packs["genemsig"],
                          p["genemsig"], NG, NGp, NMp, 1)
        r1, r2, wr = _gat(z["gene"], z["reactome"], packs["genereact"],
                          p["genereact"], NG, NGp, NRp, 1)
        bpo = _sage(z["gene"], z["bp"], packs["genebp"], p["genebp"], NBp, 2,
                    relu=True)
        gene_new = tc_combine_gene(
            gcn, a1, a2, wa, p["rev_genemsig"]["b"].reshape(1, H),
            c1, c2, wc, p["rev_genereact"]["b"].reshape(1, H), sg)
        z = {"gene": gene_new,
             "msig": tc_gat_fin(m1, m2, wm,
                                p["genemsig"]["b"].reshape(1, H), relu=True),
             "reactome": tc_gat_fin(r1, r2, wr,
                                    p["genereact"]["b"].reshape(1, H),
                                    relu=True),
             "bp": bpo}

    rel_keys = [("gene_gene", "gene", "gene"), ("gene_msig", "gene", "msig"),
                ("gene_reactome", "gene", "reactome"),
                ("gene_bp", "gene", "bp"), ("msig_gene", "msig", "gene"),
                ("reactome_gene", "reactome", "gene"),
                ("bp_gene", "bp", "gene")]
    els = {"gene_gene": el_gene_gene, "gene_msig": el_gene_msig,
           "gene_reactome": el_gene_reactome, "gene_bp": el_gene_bp,
           "msig_gene": el_msig_gene, "reactome_gene": el_reactome_gene,
           "bp_gene": el_bp_gene}
    E_LBL = el_gene_gene.shape[1]
    scores = []
    for key, st, dt in rel_keys:
        el = els[key]
        pd = params["dec"][key]
        rs = sc_gather_rows(z[st], _pad_const(el[0], elp, 0))
        rd = sc_gather_rows(z[dt], _pad_const(el[1], elp, 0))
        sc = tc_dec(rs, rd, pd["W1"][:H], pd["W1"][H:],
                    pd["b1"].reshape(1, H), pd["W2"].reshape(1, H),
                    pd["b2"].reshape(1, 1))
        scores.append(sc[:E_LBL])

    return (z["gene"][:NG], z["msig"][:NM], z["reactome"][:NR],
            z["bp"][:NB]) + tuple(scores)


# speedup vs baseline: 1.0814x; 1.0495x over previous
import functools

import jax
import jax.numpy as jnp
from jax import lax
from jax.experimental import pallas as pl
from jax.experimental.pallas import tpu as pltpu
from jax.experimental.pallas import tpu_sc as plsc

H = 128
NC, NS, L = 2, 16, 16
NW = NC * NS
WAVE = 128
EALIGN = NW * WAVE
GROWS = 256
NSLOPE = 0.2

f32 = jnp.float32
i32 = jnp.int32


def _rup(n, m):
    return ((n + m - 1) // m) * m


def _mesh():
    return plsc.VectorSubcoreMesh(core_axis_name="c", subcore_axis_name="s",
                                  num_cores=NC, num_subcores=NS)


_BR = 256


def _tc_call(body, nrows, out_width, n_in, extra_specs, out_shapes=None):
    grid = (nrows // _BR,)
    if out_shapes is None:
        out_shapes = jax.ShapeDtypeStruct((nrows, out_width), f32)
        out_specs = pl.BlockSpec((_BR, out_width), lambda i: (i, 0))
    else:
        out_specs = [pl.BlockSpec((_BR, s.shape[1]), lambda i: (i, 0))
                     for s in out_shapes]
    return pl.pallas_call(
        body, grid=grid,
        in_specs=extra_specs,
        out_specs=out_specs,
        out_shape=out_shapes,
    )


def tc_mm(x, w, b):
    def body(x_ref, w_ref, b_ref, o_ref):
        o_ref[...] = jnp.dot(x_ref[...], w_ref[...],
                             preferred_element_type=f32) + b_ref[...]
    N = x.shape[0]
    specs = [pl.BlockSpec((_BR, H), lambda i: (i, 0)),
             pl.BlockSpec((H, H), lambda i: (0, 0)),
             pl.BlockSpec((1, H), lambda i: (0, 0))]
    return _tc_call(body, N, H, 3, specs)(x, w, b)


def tc_mm2(x, y, wx, wy, b, relu):
    def body(x_ref, y_ref, wx_ref, wy_ref, b_ref, o_ref):
        acc = jnp.dot(x_ref[...], wx_ref[...], preferred_element_type=f32)
        acc = acc + jnp.dot(y_ref[...], wy_ref[...],
                            preferred_element_type=f32) + b_ref[...]
        if relu:
            acc = jnp.maximum(acc, 0.0)
        o_ref[...] = acc
    N = x.shape[0]
    specs = [pl.BlockSpec((_BR, H), lambda i: (i, 0)),
             pl.BlockSpec((_BR, H), lambda i: (i, 0)),
             pl.BlockSpec((H, H), lambda i: (0, 0)),
             pl.BlockSpec((H, H), lambda i: (0, 0)),
             pl.BlockSpec((1, H), lambda i: (0, 0))]
    return _tc_call(body, N, H, 5, specs)(x, y, wx, wy, b)


def tc_mm_rowdot(x, w, v):
    def body(x_ref, w_ref, v_ref, h_ref, e_ref):
        hh = jnp.dot(x_ref[...], w_ref[...], preferred_element_type=f32)
        h_ref[...] = hh
        e_ref[...] = jnp.sum(hh * v_ref[...], axis=1, keepdims=True)
    N = x.shape[0]
    specs = [pl.BlockSpec((_BR, H), lambda i: (i, 0)),
             pl.BlockSpec((H, H), lambda i: (0, 0)),
             pl.BlockSpec((1, H), lambda i: (0, 0))]
    outs = (jax.ShapeDtypeStruct((N, H), f32),
            jax.ShapeDtypeStruct((N, 1), f32))
    return _tc_call(body, N, None, 3, specs, out_shapes=outs)(x, w, v)


def tc_mv(x, w, v):
    def body(x_ref, w_ref, v_ref, o_ref):
        hh = jnp.dot(x_ref[...], w_ref[...], preferred_element_type=f32)
        o_ref[...] = jnp.sum(hh * v_ref[...], axis=1, keepdims=True)
    N = x.shape[0]
    specs = [pl.BlockSpec((_BR, H), lambda i: (i, 0)),
             pl.BlockSpec((H, H), lambda i: (0, 0)),
             pl.BlockSpec((1, H), lambda i: (0, 0))]
    return _tc_call(body, N, 1, 3, specs)(x, w, v)


def tc_gcn_pre(h, d0, d1):
    def body(h_ref, d0_ref, d1_ref, hp_ref, di_ref):
        dinv = lax.rsqrt(d0_ref[...] + d1_ref[...] + 1.0)
        di_ref[...] = dinv
        hp_ref[...] = h_ref[...] * dinv
    N = h.shape[0]
    specs = [pl.BlockSpec((_BR, H), lambda i: (i, 0)),
             pl.BlockSpec((_BR, 1), lambda i: (i, 0)),
             pl.BlockSpec((_BR, 1), lambda i: (i, 0))]
    outs = (jax.ShapeDtypeStruct((N, H), f32),
            jax.ShapeDtypeStruct((N, 1), f32))
    return _tc_call(body, N, None, 3, specs, out_shapes=outs)(h, d0, d1)


def tc_gcn_fin(ga, gb, dinv, h, b):
    def body(ga_ref, gb_ref, di_ref, h_ref, b_ref, o_ref):
        di = di_ref[...]
        o_ref[...] = (ga_ref[...] + gb_ref[...]) * di \
            + h_ref[...] * (di * di) + b_ref[...]
    N = h.shape[0]
    specs = [pl.BlockSpec((_BR, H), lambda i: (i, 0)),
             pl.BlockSpec((_BR, H), lambda i: (i, 0)),
             pl.BlockSpec((_BR, 1), lambda i: (i, 0)),
             pl.BlockSpec((_BR, H), lambda i: (i, 0)),
             pl.BlockSpec((1, H), lambda i: (0, 0))]
    return _tc_call(body, N, H, 5, specs)(ga, gb, dinv, h, b)


def tc_gat_fin(oa, ob, winv, b, relu):
    def body(oa_ref, ob_ref, w_ref, b_ref, o_ref):
        acc = (oa_ref[...] + ob_ref[...]) * w_ref[...] + b_ref[...]
        if relu:
            acc = jnp.maximum(acc, 0.0)
        o_ref[...] = acc
    N = oa.shape[0]
    specs = [pl.BlockSpec((_BR, H), lambda i: (i, 0)),
             pl.BlockSpec((_BR, H), lambda i: (i, 0)),
             pl.BlockSpec((_BR, 1), lambda i: (i, 0)),
             pl.BlockSpec((1, H), lambda i: (0, 0))]
    return _tc_call(body, N, H, 4, specs)(oa, ob, winv, b)


def tc_winv(d0, d1):
    def body(d0_ref, d1_ref, o_ref):
        o_ref[...] = 1.0 / (d0_ref[...] + d1_ref[...] + 1e-16)
    N = d0.shape[0]
    specs = [pl.BlockSpec((_BR, 1), lambda i: (i, 0)),
             pl.BlockSpec((_BR, 1), lambda i: (i, 0))]
    return _tc_call(body, N, 1, 2, specs)(d0, d1)


def tc_scale_rows(rows, ex):
    def body(r_ref, e_ref, o_ref):
        o_ref[...] = r_ref[...] * e_ref[...]
    N = rows.shape[0]
    specs = [pl.BlockSpec((_BR, H), lambda i: (i, 0)),
             pl.BlockSpec((_BR, 1), lambda i: (i, 0))]
    return _tc_call(body, N, H, 2, specs)(rows, ex)


def tc_sage_mean(sa, sb, c0, c1):
    def body(sa_ref, sb_ref, c0_ref, c1_ref, o_ref):
        cnt = jnp.maximum(c0_ref[...] + c1_ref[...], 1.0)
        o_ref[...] = (sa_ref[...] + sb_ref[...]) / cnt
    N = sa.shape[0]
    specs = [pl.BlockSpec((_BR, H), lambda i: (i, 0)),
             pl.BlockSpec((_BR, H), lambda i: (i, 0)),
             pl.BlockSpec((_BR, 1), lambda i: (i, 0)),
             pl.BlockSpec((_BR, 1), lambda i: (i, 0))]
    return _tc_call(body, N, H, 4, specs)(sa, sb, c0, c1)


def tc_combine_gene(gcn, a1, a2, wa, b1, c1, c2, wc, b2, sg):
    def body(g_ref, a1_ref, a2_ref, wa_ref, b1_ref, c1_ref, c2_ref, wc_ref,
             b2_ref, s_ref, o_ref):
        acc = g_ref[...] + (a1_ref[...] + a2_ref[...]) * wa_ref[...] \
            + b1_ref[...]
        acc = acc + (c1_ref[...] + c2_ref[...]) * wc_ref[...] + b2_ref[...] \
            + s_ref[...]
        o_ref[...] = jnp.maximum(acc, 0.0)
    N = gcn.shape[0]
    row = pl.BlockSpec((_BR, H), lambda i: (i, 0))
    col = pl.BlockSpec((_BR, 1), lambda i: (i, 0))
    bias = pl.BlockSpec((1, H), lambda i: (0, 0))
    specs = [row, row, row, col, bias, row, row, col, bias, row]
    return _tc_call(body, N, H, 10, specs)(gcn, a1, a2, wa, b1, c1, c2, wc,
                                           b2, sg)


def tc_dec(rs, rd, w1a, w1b, b1, w2row, b2):
    def body(rs_ref, rd_ref, wa_ref, wb_ref, b1_ref, w2_ref, b2_ref, o_ref):
        hh = jnp.dot(rs_ref[...], wa_ref[...], preferred_element_type=f32)
        hh = hh + jnp.dot(rd_ref[...], wb_ref[...], preferred_element_type=f32)
        hh = jnp.maximum(hh + b1_ref[...], 0.0)
        o_ref[...] = jnp.sum(hh * w2_ref[...], axis=1, keepdims=True) \
            + b2_ref[...]
    N = rs.shape[0]
    specs = [pl.BlockSpec((_BR, H), lambda i: (i, 0)),
             pl.BlockSpec((_BR, H), lambda i: (i, 0)),
             pl.BlockSpec((H, H), lambda i: (0, 0)),
             pl.BlockSpec((H, H), lambda i: (0, 0)),
             pl.BlockSpec((1, H), lambda i: (0, 0)),
             pl.BlockSpec((1, H), lambda i: (0, 0)),
             pl.BlockSpec((1, 1), lambda i: (0, 0))]
    return _tc_call(body, N, 1, 7, specs)(rs, rd, w1a, w1b, b1, w2row, b2)


def _zero_vec(ref, n):
    z = jnp.zeros((L,), f32)
    for k in range(n // L):
        ref[pl.ds(k * L, L)] = z


def sc_gather_rows(table, idx):
    Ep = idx.shape[0]
    ept = Ep // NW
    nwaves = ept // WAVE

    def body(table_h, idx_h, out_h, idx_v, rows_v, sem0, sem1):
        cid = lax.axis_index("c")
        sid = lax.axis_index("s")
        base = (sid * NC + cid) * ept
        sems = (sem0, sem1)

        def issue(w):
            p = w & 1
            off = base + w * WAVE
            pltpu.sync_copy(idx_h.at[pl.ds(off, WAVE)], idx_v.at[p])
            return pltpu.async_copy(table_h.at[idx_v.at[p]], rows_v.at[p],
                                    sems[p])

        h_prev = issue(0)
        for w in range(1, nwaves):
            h_cur = issue(w)
            h_prev.wait()
            pltpu.sync_copy(rows_v.at[(w - 1) & 1],
                            out_h.at[pl.ds(base + (w - 1) * WAVE, WAVE)])
            h_prev = h_cur
        h_prev.wait()
        pltpu.sync_copy(rows_v.at[(nwaves - 1) & 1],
                        out_h.at[pl.ds(base + (nwaves - 1) * WAVE, WAVE)])

    return pl.kernel(
        body,
        out_type=jax.ShapeDtypeStruct((Ep, H), f32),
        mesh=_mesh(),
        scratch_types=[pltpu.VMEM((2, WAVE), i32),
                       pltpu.VMEM((2, WAVE, H), f32),
                       pltpu.SemaphoreType.DMA,
                       pltpu.SemaphoreType.DMA],
    )(table, idx)


def sc_scatter_cols(table, src, dst, ndp, ncol):
    CW = H // ncol
    Np = table.shape[0]
    slabs = [table[:, c * CW:(c + 1) * CW] for c in range(ncol)]
    Ep = dst.shape[0]
    half = Ep // NC
    ept = half // NS
    nwaves = ept // WAVE
    stripe = ndp // NS
    assert stripe % 16 == 0
    BB = max(b for b in range(16, 129, 16) if stripe % b == 0)
    nbb = stripe // BB
    indirect = src is not None
    ins = tuple(slabs) + ((src,) if indirect else ()) + (dst,)

    def body(*refs):
        slab_hs = refs[:ncol]
        k = ncol
        if indirect:
            src_h = refs[k]
            k += 1
        dst_h = refs[k]
        out_h = refs[k + 1]
        sidx_v, loc2_v, rows_v, bounce_v, acc_sh, sem = refs[k + 2:]
        cid = lax.axis_index("c")
        sid = lax.axis_index("s")
        z16 = jnp.zeros((L,), f32)
        srow0 = sid * stripe
        tbase = cid * half + sid * ept
        if indirect:
            pltpu.sync_copy(src_h.at[pl.ds(tbase, ept)], sidx_v)
        def bw(w, _):
            pltpu.sync_copy(dst_h.at[pl.ds(tbase + w * WAVE, WAVE)],
                            loc2_v.at[w])
            for j in range(WAVE // L):
                d16 = loc2_v[w, pl.ds(j * L, L)]
                garb = ndp + (d16 & (GROWS - 1))
                loc2_v[w, pl.ds(j * L, L)] = jnp.where(d16 >= 0, d16, garb)
            return 0

        lax.fori_loop(0, nwaves, bw, 0)

        for c in range(ncol):
            for r in range(BB):
                for j in range(CW // L):
                    bounce_v[r, pl.ds(j * L, L)] = z16

            def zloop(r, _):
                pltpu.sync_copy(bounce_v,
                                acc_sh.at[pl.ds(srow0 + r * BB, BB)])
                return 0

            lax.fori_loop(0, nbb, zloop, 0)
            plsc.subcore_barrier()

            def wave(w, _):
                if indirect:
                    pltpu.async_copy(
                        slab_hs[c].at[sidx_v.at[pl.ds(w * WAVE, WAVE)]],
                        rows_v, sem).wait()
                else:
                    pltpu.sync_copy(
                        slab_hs[c].at[pl.ds(tbase + w * WAVE, WAVE)], rows_v)
                pltpu.sync_copy(rows_v, acc_sh.at[loc2_v.at[w]], add=True)
                return 0

            lax.fori_loop(0, nwaves, wave, 0)
            plsc.subcore_barrier()

            def oloop(r, _):
                pltpu.sync_copy(acc_sh.at[pl.ds(srow0 + r * BB, BB)],
                                bounce_v)
                pltpu.sync_copy(bounce_v,
                                out_h.at[cid, c,
                                         pl.ds(srow0 + r * BB, BB)])
                return 0

            lax.fori_loop(0, nbb, oloop, 0)
            plsc.subcore_barrier()

    scratch = [pltpu.VMEM((ept if indirect else L,), i32),
               pltpu.VMEM((nwaves, WAVE), i32),
               pltpu.VMEM((WAVE, CW), f32),
               pltpu.VMEM((BB, CW), f32),
               pltpu.VMEM_SHARED((ndp + GROWS, CW), f32),
               pltpu.SemaphoreType.DMA]
    out = pl.kernel(
        body,
        out_type=jax.ShapeDtypeStruct((NC, ncol, ndp, CW), f32),
        mesh=_mesh(),
        scratch_types=scratch,
        compiler_params=pltpu.CompilerParams(use_tc_tiling_on_sc=False),
    )(*ins)
    return out.transpose(0, 2, 1, 3).reshape(NC, ndp, H)


def _zero_spmem_1d(zvec_v, sh, s0, n):
    nbig, rem = divmod(n, 1024)
    for k in range(nbig):
        pltpu.sync_copy(zvec_v, sh.at[pl.ds(s0 + k * 1024, 1024)])
    if rem:
        pltpu.sync_copy(zvec_v.at[pl.ds(0, rem)],
                        sh.at[pl.ds(s0 + nbig * 1024, rem)])


def sc_gat_edge(es, ed, src, dst, ndp):
    Ep = src.shape[0]
    ept = Ep // NW
    nwaves = ept // WAVE
    stripe = ndp // NS

    def body(es_h, ed_h, src_h, dst_h, ex_h, den_h,
             sidx_v, didx_v, esg_v, edg_v, exw_v, zvec_v, dstripe_v,
             den_sh, sem):
        cid = lax.axis_index("c")
        sid = lax.axis_index("s")
        base = (sid * NC + cid) * ept
        _zero_vec(zvec_v, 1024)
        _zero_spmem_1d(zvec_v, den_sh, sid * stripe, stripe)
        plsc.subcore_barrier()

        def wave(w, _):
            off = base + w * WAVE
            pltpu.sync_copy(src_h.at[pl.ds(off, WAVE)], sidx_v)
            pltpu.sync_copy(dst_h.at[pl.ds(off, WAVE)], didx_v)
            pltpu.async_copy(es_h.at[sidx_v], esg_v, sem).wait()
            pltpu.async_copy(ed_h.at[didx_v], edg_v, sem).wait()
            for j in range(WAVE // L):
                ge = esg_v[pl.ds(j * L, L)] + edg_v[pl.ds(j * L, L)]
                ge = jnp.where(ge > 0, ge, NSLOPE * ge)
                exw_v[pl.ds(j * L, L)] = jnp.exp(ge)
            pltpu.sync_copy(exw_v, ex_h.at[pl.ds(off, WAVE)])
            pltpu.sync_copy(exw_v, den_sh.at[didx_v], add=True)
            return 0

        lax.fori_loop(0, nwaves, wave, 0)
        plsc.subcore_barrier()
        pltpu.sync_copy(den_sh.at[pl.ds(sid * stripe, stripe)], dstripe_v)
        pltpu.sync_copy(dstripe_v,
                        den_h.at[pl.ds(cid * ndp + sid * stripe, stripe)])

    return pl.kernel(
        body,
        out_type=(jax.ShapeDtypeStruct((Ep,), f32),
                  jax.ShapeDtypeStruct((NC * ndp,), f32)),
        mesh=_mesh(),
        scratch_types=[pltpu.VMEM((WAVE,), i32),
                       pltpu.VMEM((WAVE,), i32),
                       pltpu.VMEM((WAVE,), f32),
                       pltpu.VMEM((WAVE,), f32),
                       pltpu.VMEM((WAVE,), f32),
                       pltpu.VMEM((1024,), f32),
                       pltpu.VMEM((stripe,), f32),
                       pltpu.VMEM_SHARED((ndp,), f32),
                       pltpu.SemaphoreType.DMA],
    )(es, ed, src, dst)


def sc_gat_edge2(es, ed, src, dst, ndp):
    ex, den = sc_gat_edge(es, ed, src, dst, ndp)
    return ex, den.reshape(NC, ndp)


def sc_gather_scalar(vec, idx):
    Ep = idx.shape[0]
    ept = Ep // NW
    nwaves = ept // WAVE

    def body(vec_h, idx_h, out_h, idx_v, ob_v, sem):
        cid = lax.axis_index("c")
        sid = lax.axis_index("s")
        base = (sid * NC + cid) * ept

        def wave(w, _):
            off = base + w * WAVE
            pltpu.sync_copy(idx_h.at[pl.ds(off, WAVE)], idx_v)
            pltpu.async_copy(vec_h.at[idx_v], ob_v, sem).wait()
            pltpu.sync_copy(ob_v, out_h.at[pl.ds(off, WAVE)])
            return 0

        lax.fori_loop(0, nwaves, wave, 0)

    return pl.kernel(
        body,
        out_type=jax.ShapeDtypeStruct((Ep,), f32),
        mesh=_mesh(),
        scratch_types=[pltpu.VMEM((WAVE,), i32),
                       pltpu.VMEM((WAVE,), f32),
                       pltpu.SemaphoreType.DMA],
    )(vec, idx)


def sc_gather_rows2(ta, ia, tb, ib):
    Ep = ia.shape[0]
    ept = Ep // NW
    nwaves = ept // WAVE

    def body(ta_h, ia_h, tb_h, ib_h, oa_h, ob_h,
             ia_v, ib_v, ra_v, rb_v, sema, semb):
        cid = lax.axis_index("c")
        sid = lax.axis_index("s")
        base = (sid * NC + cid) * ept

        def wave(w, _):
            off = base + w * WAVE
            pltpu.sync_copy(ia_h.at[pl.ds(off, WAVE)], ia_v)
            pltpu.sync_copy(ib_h.at[pl.ds(off, WAVE)], ib_v)
            ha = pltpu.async_copy(ta_h.at[ia_v], ra_v, sema)
            hb = pltpu.async_copy(tb_h.at[ib_v], rb_v, semb)
            ha.wait()
            pltpu.sync_copy(ra_v, oa_h.at[pl.ds(off, WAVE)])
            hb.wait()
            pltpu.sync_copy(rb_v, ob_h.at[pl.ds(off, WAVE)])
            return 0

        lax.fori_loop(0, nwaves, wave, 0)

    return pl.kernel(
        body,
        out_type=(jax.ShapeDtypeStruct((Ep, H), f32),
                  jax.ShapeDtypeStruct((Ep, H), f32)),
        mesh=_mesh(),
        scratch_types=[pltpu.VMEM((WAVE,), i32),
                       pltpu.VMEM((WAVE,), i32),
                       pltpu.VMEM((WAVE, H), f32),
                       pltpu.VMEM((WAVE, H), f32),
                       pltpu.SemaphoreType.DMA,
                       pltpu.SemaphoreType.DMA],
    )(ta, ia, tb, ib)


def sc_count(dst, ndp):
    Ep = dst.shape[0]
    ept = Ep // NW
    nwaves = ept // WAVE
    stripe = ndp // NS

    def body(dst_h, den_h, didx_v, ones_v, zvec_v, dstripe_v, den_sh):
        cid = lax.axis_index("c")
        sid = lax.axis_index("s")
        base = (sid * NC + cid) * ept
        one = jnp.ones((L,), f32)
        for k in range(WAVE // L):
            ones_v[pl.ds(k * L, L)] = one
        _zero_vec(zvec_v, 1024)
        _zero_spmem_1d(zvec_v, den_sh, sid * stripe, stripe)
        plsc.subcore_barrier()

        def wave(w, _):
            off = base + w * WAVE
            pltpu.sync_copy(dst_h.at[pl.ds(off, WAVE)], didx_v)
            pltpu.sync_copy(ones_v, den_sh.at[didx_v], add=True)
            return 0

        lax.fori_loop(0, nwaves, wave, 0)
        plsc.subcore_barrier()
        pltpu.sync_copy(den_sh.at[pl.ds(sid * stripe, stripe)], dstripe_v)
        pltpu.sync_copy(dstripe_v,
                        den_h.at[pl.ds(cid * ndp + sid * stripe, stripe)])

    return pl.kernel(
        body,
        out_type=jax.ShapeDtypeStruct((NC * ndp,), f32),
        mesh=_mesh(),
        scratch_types=[pltpu.VMEM((WAVE,), i32),
                       pltpu.VMEM((WAVE,), f32),
                       pltpu.VMEM((1024,), f32),
                       pltpu.VMEM((stripe,), f32),
                       pltpu.VMEM_SHARED((ndp,), f32)],
    )(dst).reshape(NC, ndp)


def _pad_rows(x, np_):
    return jnp.pad(x, ((0, np_ - x.shape[0]), (0, 0)))


def _pad_const(v, ep, fill):
    return jnp.pad(v, (0, ep - v.shape[0]), constant_values=fill)


def _pad_spread(v, ep, base, span):
    npad = ep - v.shape[0]
    tail = base + (jnp.arange(npad, dtype=i32) % span)
    return jnp.concatenate([v, tail])


def _edges_scatter(ei, ep):
    return (_pad_const(ei[0], ep, 0),
            _pad_spread(ei[1], ep, -GROWS, GROWS))


def _mask_tail(e_col, n):
    idx = jnp.arange(e_col.shape[0], dtype=i32)[:, None]
    return jnp.where(idx < n, e_col, -1e30)


def _gcn(z, ei_pack, p, ngp):
    src0, dstn, dstc = ei_pack
    h = tc_mm(z, p["W"], jnp.zeros((1, H), f32))
    degp = sc_count(dstc, ngp)
    hpre, dinv = tc_gcn_pre(h, degp[0][:, None], degp[1][:, None])
    gp = sc_scatter_cols(hpre, src0, dstn, ngp, 4)
    return tc_gcn_fin(gp[0], gp[1], dinv, h, p["b"].reshape(1, H))


def _gat(zs, zd, pack, p, ns, nsp, ndp, ncol):
    src0, srcs, dstd, dstn = pack
    hs, es = tc_mm_rowdot(zs, p["Ws"], p["as_"].reshape(1, H))
    ed = tc_mv(zd, p["Wd"], p["ad"].reshape(1, H))
    es_m = _mask_tail(es, ns).reshape(-1)
    exv, denp = sc_gat_edge2(es_m, ed.reshape(-1), srcs, dstd, ndp)
    winv = tc_winv(denp[0][:, None], denp[1][:, None])
    rows = sc_gather_rows(hs, src0)
    msg = tc_scale_rows(rows, exv[:, None])
    op_ = sc_scatter_cols(msg, None, dstn, ndp, ncol)
    return op_[0], op_[1], winv


def _sage(zs, zd, pack, p, ndp, ncol, relu):
    src0, dstn, dstc = pack
    sp = sc_scatter_cols(zs, src0, dstn, ndp, ncol)
    cp = sc_count(dstc, ndp)
    mean = tc_sage_mean(sp[0], sp[1], cp[0][:, None], cp[1][:, None])
    return tc_mm2(mean, zd, p["Wl"], p["Wr"], p["bl"].reshape(1, H),
                  relu=relu)


def kernel(x_gene, x_msig, x_reactome, x_bp,
           ei_g2g, ei_genemsig, ei_genereact, ei_genebp,
           ei_rev_genemsig, ei_rev_genereact, ei_rev_genebp,
           el_gene_gene, el_gene_msig, el_gene_reactome, el_gene_bp,
           el_msig_gene, el_reactome_gene, el_bp_gene, params):
    NG, NM, NR, NB = (x_gene.shape[0], x_msig.shape[0],
                      x_reactome.shape[0], x_bp.shape[0])
    NGp, NMp, NRp, NBp = (_rup(NG, 256), _rup(NM, 256),
                          _rup(NR, 256), _rup(NB, 512))
    sizes = {"gene": (NG, NGp), "msig": (NM, NMp),
             "reactome": (NR, NRp), "bp": (NB, NBp)}

    z = {"gene": _pad_rows(x_gene, NGp), "msig": _pad_rows(x_msig, NMp),
         "reactome": _pad_rows(x_reactome, NRp), "bp": _pad_rows(x_bp, NBp)}

    eg = _rup(ei_g2g.shape[1], EALIGN)
    er = _rup(ei_genemsig.shape[1], EALIGN)
    elp = _rup(el_gene_gene.shape[1], EALIGN)

    g2g_pack = _edges_scatter(ei_g2g, eg) + (
        _pad_spread(ei_g2g[1], eg, NG, NGp - NG),)

    def gat_pack(ei, ns, nsp, nd, ndp):
        return (_pad_const(ei[0], er, 0),
                _pad_const(ei[0], er, ns),
                _pad_spread(ei[1], er, nd, ndp - nd),
                _pad_spread(ei[1], er, -GROWS, GROWS))

    def sage_pack(ei, nd, ndp):
        return _edges_scatter(ei, er) + (
            _pad_spread(ei[1], er, nd, ndp - nd),)

    packs = {
        "rev_genemsig": gat_pack(ei_rev_genemsig, NM, NMp, NG, NGp),
        "rev_genereact": gat_pack(ei_rev_genereact, NR, NRp, NG, NGp),
        "rev_genebp": sage_pack(ei_rev_genebp, NG, NGp),
        "genemsig": gat_pack(ei_genemsig, NG, NGp, NM, NMp),
        "genereact": gat_pack(ei_genereact, NG, NGp, NR, NRp),
        "genebp": sage_pack(ei_genebp, NB, NBp),
    }

    for p in params["layers"]:
        gcn = _gcn(z["gene"], g2g_pack, p["g2g"], NGp)
        a1, a2, wa = _gat(z["msig"], z["gene"], packs["rev_genemsig"],
                          p["rev_genemsig"], NM, NMp, NGp, 4)
        c1, c2, wc = _gat(z["reactome"], z["gene"], packs["rev_genereact"],
                          p["rev_genereact"], NR, NRp, NGp, 4)
        sg = _sage(z["bp"], z["gene"], packs["rev_genebp"], p["rev_genebp"],
                   NGp, 4, relu=False)
        m1, m2, wm = _gat(z["gene"], z["msig"], packs["genemsig"],
                          p["genemsig"], NG, NGp, NMp, 1)
        r1, r2, wr = _gat(z["gene"], z["reactome"], packs["genereact"],
                          p["genereact"], NG, NGp, NRp, 1)
        bpo = _sage(z["gene"], z["bp"], packs["genebp"], p["genebp"], NBp, 2,
                    relu=True)
        gene_new = tc_combine_gene(
            gcn, a1, a2, wa, p["rev_genemsig"]["b"].reshape(1, H),
            c1, c2, wc, p["rev_genereact"]["b"].reshape(1, H), sg)
        z = {"gene": gene_new,
             "msig": tc_gat_fin(m1, m2, wm,
                                p["genemsig"]["b"].reshape(1, H), relu=True),
             "reactome": tc_gat_fin(r1, r2, wr,
                                    p["genereact"]["b"].reshape(1, H),
                                    relu=True),
             "bp": bpo}

    rel_keys = [("gene_gene", "gene", "gene"), ("gene_msig", "gene", "msig"),
                ("gene_reactome", "gene", "reactome"),
                ("gene_bp", "gene", "bp"), ("msig_gene", "msig", "gene"),
                ("reactome_gene", "reactome", "gene"),
                ("bp_gene", "bp", "gene")]
    els = {"gene_gene": el_gene_gene, "gene_msig": el_gene_msig,
           "gene_reactome": el_gene_reactome, "gene_bp": el_gene_bp,
           "msig_gene": el_msig_gene, "reactome_gene": el_reactome_gene,
           "bp_gene": el_bp_gene}
    E_LBL = el_gene_gene.shape[1]
    scores = []
    for key, st, dt in rel_keys:
        el = els[key]
        pd = params["dec"][key]
        rs, rd = sc_gather_rows2(z[st], _pad_const(el[0], elp, 0),
                                 z[dt], _pad_const(el[1], elp, 0))
        sc = tc_dec(rs, rd, pd["W1"][:H], pd["W1"][H:],
                    pd["b1"].reshape(1, H), pd["W2"].reshape(1, H),
                    pd["b2"].reshape(1, 1))
        scores.append(sc[:E_LBL])

    return (z["gene"][:NG], z["msig"][:NM], z["reactome"][:NR],
            z["bp"][:NB]) + tuple(scores)
